# baseline (speedup 1.0000x reference)
import functools

import jax
import jax.numpy as jnp
from jax import lax
from jax.experimental import pallas as pl
from jax.experimental.pallas import tpu as pltpu
from jax.experimental.pallas import tpu_sc as plsc

N = 50000
D = 592
LANES = 16
NC = 2
NS = 16

CB = 1664
NTAIL = 80
TAIL0 = N - NTAIL
UNITS = D // 8
TROWS = D // 2
NIN = 4

_mesh = plsc.VectorSubcoreMesh(core_axis_name="c", subcore_axis_name="s")


def _seg_of_unit(u):
    if isinstance(u, int):
        return int(u >= 16) + int(u >= 40) + int(u >= 60)
    return (
        (u >= 16).astype(jnp.int32)
        + (u >= 40).astype(jnp.int32)
        + (u >= 60).astype(jnp.int32)
    )


@functools.partial(
    pl.kernel,
    mesh=_mesh,
    out_type=jax.ShapeDtypeStruct((D, N), jnp.float32),
    scratch_types=[
        pltpu.VMEM((NIN, 8, CB), jnp.float32),
        pltpu.VMEM((2, 8, CB), jnp.float32),
        pltpu.VMEM((4, CB), jnp.float32),
        pltpu.VMEM((152, NTAIL), jnp.float32),
        pltpu.VMEM((4, NTAIL), jnp.float32),
        pltpu.SemaphoreType.DMA,
        pltpu.SemaphoreType.DMA,
        pltpu.SemaphoreType.DMA,
        pltpu.SemaphoreType.DMA,
        pltpu.SemaphoreType.DMA,
        pltpu.SemaphoreType.DMA,
    ],
    compiler_params=pltpu.CompilerParams(use_tc_tiling_on_sc=True),
)
def _irrepwise_sc_t(xt_hbm, wt_hbm, out_hbm, xb, ob, wv, tb, twv,
                    sx0, sx1, sx2, sx3, so0, so1):
    wid = lax.axis_index("s") * NC + lax.axis_index("c")
    cbase = pl.multiple_of(jnp.minimum(wid, 29) * CB, 128)

    sx = (sx0, sx1, sx2, sx3)
    so = (so0, so1)

    def start_in(u, b):
        j0 = pl.multiple_of(u * 8, 8)
        pltpu.async_copy(
            xt_hbm.at[pl.ds(j0, 8), pl.ds(cbase, CB)], xb.at[b], sx[b]
        )

    def wait_in(b):
        pltpu.make_async_copy(
            xt_hbm.at[pl.ds(0, 8), pl.ds(cbase, CB)], xb.at[b], sx[b]
        ).wait()

    def start_out(u, b):
        j0 = pl.multiple_of(u * 8, 8)
        pltpu.async_copy(
            ob.at[b], out_hbm.at[pl.ds(j0, 8), pl.ds(cbase, CB)], so[b]
        )

    def wait_out(b):
        pltpu.make_async_copy(
            ob.at[b], out_hbm.at[pl.ds(0, 8), pl.ds(cbase, CB)], so[b]
        ).wait()

    def compute(u, bi, bo):
        s = _seg_of_unit(u)

        @pl.loop(0, CB // LANES)
        def _chunk(k):
            c = k * LANES
            wk = wv[s, pl.ds(c, LANES)]
            for r in range(8):
                ob[bo, r, pl.ds(c, LANES)] = xb[bi, r, pl.ds(c, LANES)] * wk

    def step(u, bi, bo, prefetch, first_out):
        wait_in(bi)
        if not first_out:
            wait_out(bo)
        compute(u, bi, bo)
        if prefetch:
            start_in(u + NIN, bi)
        start_out(u, bo)

    @pl.when(wid < 30)
    def _main():
        pltpu.sync_copy(wt_hbm.at[:, pl.ds(cbase, CB)], wv)

        for u in range(NIN):
            start_in(u, u)

        step(0, 0, 0, True, True)
        step(1, 1, 1, True, True)
        step(2, 2, 0, True, False)
        step(3, 3, 1, True, False)

        @pl.loop(1, 17)
        def _grp(i):
            for b in range(NIN):
                u = 4 * i + b
                step(u, b, b % 2, True, False)

        for u in range(68, UNITS):
            step(u, u % NIN, u % 2, u + NIN < UNITS, False)

        wait_out(0)
        wait_out(1)

    @pl.when(wid >= 30)
    def _tail():
        row0 = pl.multiple_of((wid - 30) * TROWS, 8)
        pltpu.sync_copy(wt_hbm.at[:, pl.ds(TAIL0, NTAIL)], twv)
        for off, rows in ((0, 144), (144, 152)):
            r0 = row0 + off
            u0 = (wid - 30) * (TROWS // 8) + off // 8
            tbs = tb.at[pl.ds(0, rows)]
            pltpu.sync_copy(
                xt_hbm.at[pl.ds(r0, rows), pl.ds(TAIL0, NTAIL)], tbs
            )

            @pl.loop(0, rows // 8)
            def _u(u):
                s = _seg_of_unit(u0 + u)
                j0 = u * 8
                for k in range(NTAIL // LANES):
                    c = k * LANES
                    wk = twv[s, pl.ds(c, LANES)]
                    for r in range(8):
                        tb[j0 + r, pl.ds(c, LANES)] = (
                            tb[j0 + r, pl.ds(c, LANES)] * wk
                        )

            pltpu.sync_copy(
                tbs, out_hbm.at[pl.ds(r0, rows), pl.ds(TAIL0, NTAIL)]
            )


@jax.jit
def kernel(x, w):
    return _irrepwise_sc_t(x.T, w.T).T


# speedup vs baseline: 1.2837x; 1.2837x over previous
import functools

import jax
import jax.numpy as jnp
from jax import lax
from jax.experimental import pallas as pl
from jax.experimental.pallas import tpu as pltpu
from jax.experimental.pallas import tpu_sc as plsc

N = 50000
D = 592
LANES = 16
NC = 2
NS = 16

CB = 1664
NTAIL = 80
TAIL0 = N - NTAIL
UNITS = D // 8
TROWS = D // 2
NIN = 2

_mesh = plsc.VectorSubcoreMesh(core_axis_name="c", subcore_axis_name="s")


def _seg_of_unit(u):
    if isinstance(u, int):
        return int(u >= 16) + int(u >= 40) + int(u >= 60)
    return (
        (u >= 16).astype(jnp.int32)
        + (u >= 40).astype(jnp.int32)
        + (u >= 60).astype(jnp.int32)
    )


@functools.partial(
    pl.kernel,
    mesh=_mesh,
    out_type=jax.ShapeDtypeStruct((D, N), jnp.float32),
    scratch_types=[
        pltpu.VMEM((NIN, 8, CB), jnp.float32),
        pltpu.VMEM((2, 8, CB), jnp.float32),
        pltpu.VMEM((4, CB), jnp.float32),
        pltpu.VMEM((152, NTAIL), jnp.float32),
        pltpu.VMEM((4, NTAIL), jnp.float32),
        pltpu.SemaphoreType.DMA,
        pltpu.SemaphoreType.DMA,
        pltpu.SemaphoreType.DMA,
        pltpu.SemaphoreType.DMA,
    ],
    compiler_params=pltpu.CompilerParams(use_tc_tiling_on_sc=True),
)
def _irrepwise_sc_t(xt_hbm, wt_hbm, out_hbm, xb, ob, wv, tb, twv,
                    sx0, sx1, so0, so1):
    wid = lax.axis_index("s") * NC + lax.axis_index("c")
    cbase = pl.multiple_of(jnp.minimum(wid, 29) * CB, 128)

    sx = (sx0, sx1)
    so = (so0, so1)

    def start_in(u, b):
        j0 = pl.multiple_of(u * 8, 8)
        pltpu.async_copy(
            xt_hbm.at[pl.ds(j0, 8), pl.ds(cbase, CB)], xb.at[b], sx[b]
        )

    def wait_in(b):
        pltpu.make_async_copy(
            xt_hbm.at[pl.ds(0, 8), pl.ds(cbase, CB)], xb.at[b], sx[b]
        ).wait()

    def start_out(u, b):
        j0 = pl.multiple_of(u * 8, 8)
        pltpu.async_copy(
            ob.at[b], out_hbm.at[pl.ds(j0, 8), pl.ds(cbase, CB)], so[b]
        )

    def wait_out(b):
        pltpu.make_async_copy(
            ob.at[b], out_hbm.at[pl.ds(0, 8), pl.ds(cbase, CB)], so[b]
        ).wait()

    def compute(u, bi, bo):
        s = _seg_of_unit(u)

        @pl.loop(0, CB // LANES)
        def _chunk(k):
            c = k * LANES
            wk = wv[s, pl.ds(c, LANES)]
            for r in range(8):
                ob[bo, r, pl.ds(c, LANES)] = xb[bi, r, pl.ds(c, LANES)] * wk

    def step(u, bi, bo, prefetch, first_out):
        wait_in(bi)
        if not first_out:
            wait_out(bo)
        compute(u, bi, bo)
        if prefetch:
            start_in(u + NIN, bi)
        start_out(u, bo)

    @pl.when(wid < 30)
    def _main():
        pltpu.sync_copy(wt_hbm.at[:, pl.ds(cbase, CB)], wv)

        for u in range(NIN):
            start_in(u, u)

        step(0, 0, 0, True, True)
        step(1, 1, 1, True, True)

        @pl.loop(1, UNITS // 2 - 1)
        def _pair(i):
            for b in (0, 1):
                u = 2 * i + b
                step(u, b, b, True, False)

        step(UNITS - 2, 0, 0, False, False)
        step(UNITS - 1, 1, 1, False, False)

        wait_out(0)
        wait_out(1)

    @pl.when(wid >= 30)
    def _tail():
        row0 = pl.multiple_of((wid - 30) * TROWS, 8)
        pltpu.sync_copy(wt_hbm.at[:, pl.ds(TAIL0, NTAIL)], twv)
        for off, rows in ((0, 144), (144, 152)):
            r0 = row0 + off
            u0 = (wid - 30) * (TROWS // 8) + off // 8
            tbs = tb.at[pl.ds(0, rows)]
            pltpu.sync_copy(
                xt_hbm.at[pl.ds(r0, rows), pl.ds(TAIL0, NTAIL)], tbs
            )

            @pl.loop(0, rows // 8)
            def _u(u):
                s = _seg_of_unit(u0 + u)
                j0 = u * 8
                for k in range(NTAIL // LANES):
                    c = k * LANES
                    wk = twv[s, pl.ds(c, LANES)]
                    for r in range(8):
                        tb[j0 + r, pl.ds(c, LANES)] = (
                            tb[j0 + r, pl.ds(c, LANES)] * wk
                        )

            pltpu.sync_copy(
                tbs, out_hbm.at[pl.ds(r0, rows), pl.ds(TAIL0, NTAIL)]
            )


@jax.jit
def kernel(x, w):
    return _irrepwise_sc_t(x.T, w.T).T
